# Initial kernel scaffold; baseline (speedup 1.0000x reference)
import jax
import jax.numpy as jnp
from jax.experimental import pallas as pl


def kernel(logits, labels):
    raise NotImplementedError("write your pallas kernel here")


# speedup vs baseline: 8.6847x; 8.6847x over previous
import functools

import jax
import jax.numpy as jnp
from jax import lax
from jax.experimental import pallas as pl
from jax.experimental.pallas import tpu as pltpu
from jax.experimental.pallas import tpu_sc as plsc

_N = 16 * 512 * 512
_NC, _NS = 2, 16
_NW = _NC * _NS
_Q = _N // _NW
_S = 2048
_SUB = 128
_BATCH = 1024
_ROWS_B = _BATCH // _SUB
_SHIFT = 12
_B = 1 << 19
_Z = _B // _NS
_ROWS = _B // 128
_BLK_R = 512
_NBLK = _ROWS // _BLK_R


def _hist_body(log_hbm, lab_hbm, hp_out, hc_out, hs_out,
               sh_hp, sh_hc, sh_hs,
               log_v, lab_v, idx_v, val_v, pv_v, ones_v):
    cid = lax.axis_index("c")
    sid = lax.axis_index("s")
    wid = cid * _NS + sid

    def _zero16(i, _):
        log_v[pl.ds(i * 16, 16)] = jnp.zeros((16,), jnp.float32)
        return 0
    lax.fori_loop(0, _S // 16, _zero16, 0)
    def _ones16(i, _):
        ones_v[pl.ds(i * 16, 16)] = jnp.ones((16,), jnp.float32)
        return 0
    lax.fori_loop(0, _BATCH // 16, _ones16, 0)

    def _zero_hist(i, _):
        off = sid * _Z + i * _S
        pltpu.sync_copy(log_v, sh_hp.at[pl.ds(off, _S)])
        pltpu.sync_copy(log_v, sh_hc.at[pl.ds(off, _S)])
        pltpu.sync_copy(log_v, sh_hs.at[pl.ds(off, _S)])
        return 0
    lax.fori_loop(0, _Z // _S, _zero_hist, 0)
    plsc.subcore_barrier()

    base = wid * _Q

    def _outer(t, _):
        b0 = base + t * _S
        pltpu.sync_copy(log_hbm.at[pl.ds(b0, _S)], log_v)
        pltpu.sync_copy(lab_hbm.at[pl.ds(b0, _S)], lab_v)

        def _inner(k, _):
            off = k * _BATCH
            for j in range(_BATCH // 16):
                o = off + j * 16
                s = log_v[pl.ds(o, 16)]
                l = lab_v[pl.ds(o, 16)]
                e = 1.0 - s * (2.0 * l - 1.0)
                bits = lax.bitcast_convert_type(e, jnp.int32)
                key = jnp.where(e > 0.0,
                                lax.shift_right_logical(bits, _SHIFT),
                                0)
                idx_v[pl.ds(j * 16, 16)] = key
                val_v[pl.ds(j * 16, 16)] = jnp.maximum(e, 0.0)
                pv_v[pl.ds(j * 16, 16)] = l
            pltpu.sync_copy(val_v, sh_hs.at[idx_v], add=True)
            pltpu.sync_copy(pv_v, sh_hp.at[idx_v], add=True)
            pltpu.sync_copy(ones_v, sh_hc.at[idx_v], add=True)
            return 0
        lax.fori_loop(0, _S // _BATCH, _inner, 0)
        return 0
    lax.fori_loop(0, _Q // _S, _outer, 0)
    plsc.subcore_barrier()

    z0 = sid * _Z
    pltpu.sync_copy(sh_hp.at[pl.ds(z0, _Z)], hp_out.at[cid, pl.ds(z0, _Z)])
    pltpu.sync_copy(sh_hc.at[pl.ds(z0, _Z)], hc_out.at[cid, pl.ds(z0, _Z)])
    pltpu.sync_copy(sh_hs.at[pl.ds(z0, _Z)], hs_out.at[cid, pl.ds(z0, _Z)])


def _make_hist():
  return functools.partial(
    pl.kernel,
    out_type=[jax.ShapeDtypeStruct((_NC, _B), jnp.float32)] * 3,
    mesh=plsc.VectorSubcoreMesh(core_axis_name="c", subcore_axis_name="s",
                                num_cores=_NC, num_subcores=_NS),
    scratch_types=[
        pltpu.VMEM_SHARED((_B,), jnp.float32),
        pltpu.VMEM_SHARED((_B,), jnp.float32),
        pltpu.VMEM_SHARED((_B,), jnp.float32),
        pltpu.VMEM((_S,), jnp.float32),
        pltpu.VMEM((_S,), jnp.float32),
        pltpu.VMEM((_BATCH,), jnp.int32),
        pltpu.VMEM((_BATCH,), jnp.float32),
        pltpu.VMEM((_BATCH,), jnp.float32),
        pltpu.VMEM((_BATCH,), jnp.float32),
    ],
  )(_hist_body)


def _psum_body(hp_ref, o_ref):
    o_ref[0, 0] = jnp.sum(hp_ref[...])


def _scan_body(p_ref, hp_ref, hc_ref, hs_ref, o_ref, acc):
    i = pl.program_id(0)

    @pl.when(i == 0)
    def _():
        acc[0] = 0.0
        acc[1] = 0.0
        acc[2] = 0.0

    Xp = jnp.sum(hp_ref[...], axis=0)
    Xc = jnp.sum(hc_ref[...], axis=0)
    Xs = jnp.sum(hs_ref[...], axis=0)
    P = p_ref[0, 0]

    ci = lax.broadcasted_iota(jnp.int32, (128, 128), 0)
    cj = lax.broadcasted_iota(jnp.int32, (128, 128), 1)
    M = (ci > cj).astype(jnp.float32)
    ri = lax.broadcasted_iota(jnp.int32, (_BLK_R, _BLK_R), 0)
    rj = lax.broadcasted_iota(jnp.int32, (_BLK_R, _BLK_R), 1)
    T = (rj > ri).astype(jnp.float32)

    colsuf_p = jnp.dot(Xp, M, preferred_element_type=jnp.float32)
    colsuf_c = jnp.dot(Xc, M, preferred_element_type=jnp.float32)
    below_p = jnp.sum(jnp.dot(T, Xp, preferred_element_type=jnp.float32),
                      axis=1, keepdims=True)
    below_c = jnp.sum(jnp.dot(T, Xc, preferred_element_type=jnp.float32),
                      axis=1, keepdims=True)

    CP = colsuf_p + below_p + acc[0]
    CC = colsuf_c + below_c + acc[1]
    CN = CC - CP
    p = Xp
    c = Xc
    n = c - p
    D0 = P + CN
    D1 = D0 + n
    num = p * D0 + n * (P - CP)
    delta = jnp.where(D0 > 0.0,
                      num / jnp.maximum(D0 * D1, 1.0),
                      (p + n) / jnp.maximum(D1, 1.0))
    contrib = jnp.where(c > 0.0, Xs / jnp.maximum(c, 1.0), 0.0) * delta
    loss_b = jnp.sum(contrib)

    acc[2] = acc[2] + loss_b
    acc[0] = acc[0] + jnp.sum(Xp)
    acc[1] = acc[1] + jnp.sum(Xc)
    o_ref[0, 0] = acc[2]


def kernel(logits, labels):
    x = logits.reshape(_N)
    lab = labels.astype(jnp.float32).reshape(_N)

    hp, hc, hs = _make_hist()(x, lab)

    hp3 = hp.reshape(_NC, _ROWS, 128)
    hc3 = hc.reshape(_NC, _ROWS, 128)
    hs3 = hs.reshape(_NC, _ROWS, 128)

    ptot = pl.pallas_call(
        _psum_body,
        out_shape=jax.ShapeDtypeStruct((1, 1), jnp.float32),
        in_specs=[pl.BlockSpec((_NC, _ROWS, 128), lambda: (0, 0, 0))],
        out_specs=pl.BlockSpec(memory_space=pltpu.SMEM),
    )(hp3)

    loss = pl.pallas_call(
        _scan_body,
        grid=(_NBLK,),
        out_shape=jax.ShapeDtypeStruct((1, 1), jnp.float32),
        in_specs=[
            pl.BlockSpec(memory_space=pltpu.SMEM),
            pl.BlockSpec((_NC, _BLK_R, 128), lambda i: (0, _NBLK - 1 - i, 0)),
            pl.BlockSpec((_NC, _BLK_R, 128), lambda i: (0, _NBLK - 1 - i, 0)),
            pl.BlockSpec((_NC, _BLK_R, 128), lambda i: (0, _NBLK - 1 - i, 0)),
        ],
        out_specs=pl.BlockSpec(memory_space=pltpu.SMEM),
        scratch_shapes=[pltpu.SMEM((4,), jnp.float32)],
    )(ptot, hp3, hc3, hs3)

    return loss[0, 0]
